# baseline (speedup 1.0000x reference)
import functools

import jax
import jax.numpy as jnp
from jax import lax
from jax.experimental import pallas as pl
from jax.experimental.pallas import tpu as pltpu
from jax.experimental.pallas import tpu_sc as plsc

B, NH, V, D, HID = 4096, 20, 100000, 64, 128
N = B * NH
F = NH * D

NC, NS = 2, 16
NW = NC * NS
ROWS_W = N // NW
CH = 128
NCH = ROWS_W // CH

BV = 1024


def _gather_feat(idx_flat, table):
    mesh = plsc.VectorSubcoreMesh(core_axis_name="c", subcore_axis_name="s")

    @functools.partial(
        pl.kernel,
        mesh=mesh,
        out_type=jax.ShapeDtypeStruct((N, D), jnp.bfloat16),
        scratch_types=[
            pltpu.VMEM((ROWS_W,), jnp.int32),
            pltpu.VMEM((CH, D), jnp.bfloat16),
            pltpu.VMEM((CH, D), jnp.bfloat16),
            pltpu.SemaphoreType.DMA,
            pltpu.SemaphoreType.DMA,
        ],
        compiler_params=pltpu.CompilerParams(use_tc_tiling_on_sc=False),
    )
    def gather_kernel(idx_hbm, table_hbm, out_hbm, idx_v, rows0, rows1, g0, g1):
        wid = lax.axis_index("s") * NC + lax.axis_index("c")
        base = wid * ROWS_W
        pltpu.sync_copy(idx_hbm.at[pl.ds(base, ROWS_W)], idx_v)
        rows = (rows0, rows1)
        gsem = (g0, g1)
        pend = [None, None]
        pend[0] = pltpu.async_copy(
            table_hbm.at[idx_v.at[pl.ds(0, CH)]], rows0, g0
        )
        for c in range(NCH):
            cur = c % 2
            nxt = (c + 1) % 2
            if c + 1 < NCH:
                pend[nxt] = pltpu.async_copy(
                    table_hbm.at[idx_v.at[pl.ds((c + 1) * CH, CH)]],
                    rows[nxt],
                    gsem[nxt],
                )
            pend[cur].wait()
            pltpu.sync_copy(rows[cur], out_hbm.at[pl.ds(base + c * CH, CH)])

    return gather_kernel(idx_flat, table)


def _h_body(feat_ref, w1_ref, b1_ref, h_ref):
    h_ref[...] = jnp.tanh(
        jnp.dot(feat_ref[...], w1_ref[...].astype(jnp.bfloat16),
                preferred_element_type=jnp.float32)
        + b1_ref[...]
    )


def _h_kernel(feat, w1, b1_2d):
    return pl.pallas_call(
        _h_body,
        out_shape=jax.ShapeDtypeStruct((B, HID), jnp.float32),
    )(feat, w1, b1_2d)


def _proj_body(w2t_ref, ht_ref, b2_ref, out_ref):
    acc = jnp.dot(w2t_ref[...], ht_ref[...], preferred_element_type=jnp.float32)
    bias = lax.dot_general(
        b2_ref[...],
        jnp.ones((1, B), jnp.float32),
        (((0,), (0,)), ((), ())),
        preferred_element_type=jnp.float32,
    )
    out_ref[...] = acc + bias


def _proj(w2t, ht, b2_row):
    grid = (pl.cdiv(V, BV),)
    return pl.pallas_call(
        _proj_body,
        grid=grid,
        in_specs=[
            pl.BlockSpec((BV, HID), lambda i: (i, 0)),
            pl.BlockSpec((HID, B), lambda i: (0, 0)),
            pl.BlockSpec((1, BV), lambda i: (0, i)),
        ],
        out_specs=pl.BlockSpec((BV, B), lambda i: (i, 0)),
        out_shape=jax.ShapeDtypeStruct((V, B), jnp.float32),
        compiler_params=pltpu.CompilerParams(
            dimension_semantics=("arbitrary",),
            vmem_limit_bytes=64 * 1024 * 1024,
        ),
    )(w2t, ht, b2_row)


def kernel(input, emb_table, W1, b1, W2, b2):
    idx_flat = input.reshape(-1)
    feat = _gather_feat(idx_flat, emb_table.astype(jnp.bfloat16)).reshape(B, F)
    h = _h_kernel(feat, W1, b1.reshape(1, HID))
    out_t = _proj(W2.T, h.T, b2.reshape(1, V))
    return out_t.T


# speedup vs baseline: 1.0905x; 1.0905x over previous
import functools

import jax
import jax.numpy as jnp
from jax import lax
from jax.experimental import pallas as pl
from jax.experimental.pallas import tpu as pltpu
from jax.experimental.pallas import tpu_sc as plsc

B, NH, V, D, HID = 4096, 20, 100000, 64, 128
N = B * NH
F = NH * D

NC, NS = 2, 16
NW = NC * NS
ROWS_W = N // NW
CH = 128
NCH = ROWS_W // CH

BV = 1024


def _gather_feat(idx_flat, table):
    mesh = plsc.VectorSubcoreMesh(core_axis_name="c", subcore_axis_name="s")

    @functools.partial(
        pl.kernel,
        mesh=mesh,
        out_type=jax.ShapeDtypeStruct((N, D), jnp.float32),
        scratch_types=[
            pltpu.VMEM((ROWS_W,), jnp.int32),
            pltpu.VMEM((CH, D), jnp.float32),
            pltpu.VMEM((CH, D), jnp.float32),
            pltpu.SemaphoreType.DMA,
            pltpu.SemaphoreType.DMA,
        ],
        compiler_params=pltpu.CompilerParams(use_tc_tiling_on_sc=False),
    )
    def gather_kernel(idx_hbm, table_hbm, out_hbm, idx_v, rows0, rows1, g0, g1):
        wid = lax.axis_index("s") * NC + lax.axis_index("c")
        base = wid * ROWS_W
        pltpu.sync_copy(idx_hbm.at[pl.ds(base, ROWS_W)], idx_v)
        rows = (rows0, rows1)
        gsem = (g0, g1)
        pend = [None, None]
        pend[0] = pltpu.async_copy(
            table_hbm.at[idx_v.at[pl.ds(0, CH)]], rows0, g0
        )
        for c in range(NCH):
            cur = c % 2
            nxt = (c + 1) % 2
            if c + 1 < NCH:
                pend[nxt] = pltpu.async_copy(
                    table_hbm.at[idx_v.at[pl.ds((c + 1) * CH, CH)]],
                    rows[nxt],
                    gsem[nxt],
                )
            pend[cur].wait()
            pltpu.sync_copy(rows[cur], out_hbm.at[pl.ds(base + c * CH, CH)])

    return gather_kernel(idx_flat, table)


def _proj_body(feat_ref, w1_ref, b1c_ref, w2t_ref, b2_ref, out_ref, ht_ref):
    @pl.when(pl.program_id(0) == 0)
    def _():
        ht_ref[...] = jnp.tanh(
            lax.dot_general(
                w1_ref[...],
                feat_ref[...],
                (((0,), (1,)), ((), ())),
                preferred_element_type=jnp.float32,
            )
            + b1c_ref[...]
        )

    acc = jnp.dot(w2t_ref[...], ht_ref[...], preferred_element_type=jnp.float32)
    bias = lax.dot_general(
        b2_ref[...],
        jnp.ones((1, B), jnp.float32),
        (((0,), (0,)), ((), ())),
        preferred_element_type=jnp.float32,
    )
    out_ref[...] = acc + bias


def _proj(feat, w1, b1_col, w2t, b2_row):
    grid = (pl.cdiv(V, BV),)
    return pl.pallas_call(
        _proj_body,
        grid=grid,
        in_specs=[
            pl.BlockSpec((B, F), lambda i: (0, 0)),
            pl.BlockSpec((F, HID), lambda i: (0, 0)),
            pl.BlockSpec((HID, 1), lambda i: (0, 0)),
            pl.BlockSpec((BV, HID), lambda i: (i, 0)),
            pl.BlockSpec((1, BV), lambda i: (0, i)),
        ],
        out_specs=pl.BlockSpec((BV, B), lambda i: (i, 0)),
        out_shape=jax.ShapeDtypeStruct((V, B), jnp.float32),
        scratch_shapes=[pltpu.VMEM((HID, B), jnp.float32)],
        compiler_params=pltpu.CompilerParams(
            dimension_semantics=("arbitrary",),
            vmem_limit_bytes=64 * 1024 * 1024,
        ),
    )(feat, w1, b1_col, w2t, b2_row)


def kernel(input, emb_table, W1, b1, W2, b2):
    idx_flat = input.reshape(-1)
    feat = _gather_feat(idx_flat, emb_table).reshape(B, F)
    out_t = _proj(feat, W1, b1.reshape(HID, 1), W2.T, b2.reshape(1, V))
    return out_t.T


# speedup vs baseline: 1.0954x; 1.0045x over previous
import functools

import jax
import jax.numpy as jnp
from jax import lax
from jax.experimental import pallas as pl
from jax.experimental.pallas import tpu as pltpu
from jax.experimental.pallas import tpu_sc as plsc

B, NH, V, D, HID = 4096, 20, 100000, 64, 128
N = B * NH
F = NH * D

NC, NS = 2, 16
NW = NC * NS
ROWS_W = N // NW
CH = 128
NCH = ROWS_W // CH

BV = 1024


def _gather_feat(idx_flat, table):
    mesh = plsc.VectorSubcoreMesh(core_axis_name="c", subcore_axis_name="s")

    @functools.partial(
        pl.kernel,
        mesh=mesh,
        out_type=jax.ShapeDtypeStruct((N, D), jnp.float32),
        scratch_types=[
            pltpu.VMEM((ROWS_W,), jnp.int32),
            pltpu.VMEM((CH, D), jnp.float32),
            pltpu.VMEM((CH, D), jnp.float32),
            pltpu.SemaphoreType.DMA,
            pltpu.SemaphoreType.DMA,
        ],
        compiler_params=pltpu.CompilerParams(use_tc_tiling_on_sc=False),
    )
    def gather_kernel(idx_hbm, table_hbm, out_hbm, idx_v, rows0, rows1, g0, g1):
        wid = lax.axis_index("s") * NC + lax.axis_index("c")
        base = wid * ROWS_W
        pltpu.sync_copy(idx_hbm.at[pl.ds(base, ROWS_W)], idx_v)
        rows = (rows0, rows1)
        gsem = (g0, g1)
        pend = [None, None]
        pend[0] = pltpu.async_copy(
            table_hbm.at[idx_v.at[pl.ds(0, CH)]], rows0, g0
        )
        for c in range(NCH):
            cur = c % 2
            nxt = (c + 1) % 2
            if c + 1 < NCH:
                pend[nxt] = pltpu.async_copy(
                    table_hbm.at[idx_v.at[pl.ds((c + 1) * CH, CH)]],
                    rows[nxt],
                    gsem[nxt],
                )
            pend[cur].wait()
            pltpu.sync_copy(rows[cur], out_hbm.at[pl.ds(base + c * CH, CH)])

    return gather_kernel(idx_flat, table)


def _proj_body(feat_ref, w1_ref, b1c_ref, w2t_ref, b2_ref, out_ref, ht_ref):
    @pl.when(pl.program_id(0) == 0)
    def _():
        ht_ref[...] = jnp.tanh(
            lax.dot_general(
                w1_ref[...],
                feat_ref[...],
                (((0,), (1,)), ((), ())),
                preferred_element_type=jnp.float32,
            )
            + b1c_ref[...]
        )

    acc = jnp.dot(w2t_ref[...], ht_ref[...], preferred_element_type=jnp.float32)
    bias = lax.dot_general(
        b2_ref[...],
        jnp.ones((1, B), jnp.float32),
        (((0,), (0,)), ((), ())),
        preferred_element_type=jnp.float32,
    )
    out_ref[...] = acc + bias


def _proj(feat, w1, b1_col, w2t, b2_row):
    grid = (pl.cdiv(V, BV),)
    return pl.pallas_call(
        _proj_body,
        grid=grid,
        in_specs=[
            pl.BlockSpec((B, F), lambda i: (0, 0)),
            pl.BlockSpec((F, HID), lambda i: (0, 0)),
            pl.BlockSpec((HID, 1), lambda i: (0, 0)),
            pl.BlockSpec((BV, HID), lambda i: (i, 0)),
            pl.BlockSpec((1, BV), lambda i: (0, i)),
        ],
        out_specs=pl.BlockSpec((BV, B), lambda i: (i, 0)),
        out_shape=jax.ShapeDtypeStruct((V, B), jnp.float32),
        scratch_shapes=[pltpu.VMEM((HID, B), jnp.float32)],
        compiler_params=pltpu.CompilerParams(
            dimension_semantics=("arbitrary",),
            vmem_limit_bytes=64 * 1024 * 1024,
        ),
    )(feat, w1, b1_col, w2t, b2_row)


def kernel(input, emb_table, W1, b1, W2, b2):
    idx_flat = input.reshape(-1)
    table_lin = lax.optimization_barrier(
        emb_table.reshape(V // 2, 2 * D)
    ).reshape(V, D)
    feat = _gather_feat(idx_flat, table_lin).reshape(B, F)
    out_t = _proj(feat, W1, b1.reshape(HID, 1), W2.T, b2.reshape(1, V))
    return out_t.T
